# Initial kernel scaffold; baseline (speedup 1.0000x reference)
import functools

import jax
import jax.numpy as jnp
from jax import lax
from jax.experimental import pallas as pl
from jax.experimental.pallas import tpu as pltpu
from jax.experimental.pallas import tpu_sc as plsc

_NS = 16
_CH = 128


def _matmul_split(x, w):
    n = x.shape[0]
    h = w.shape[1] // 2

    def body(x_ref, w_ref, o0_ref, o1_ref):
        xv = x_ref[...]
        o0_ref[...] = jnp.dot(xv, w_ref[:, :h], preferred_element_type=jnp.float32)
        o1_ref[...] = jnp.dot(xv, w_ref[:, h:], preferred_element_type=jnp.float32)

    return pl.pallas_call(
        body,
        out_shape=[jax.ShapeDtypeStruct((n, h), jnp.float32)] * 2,
    )(x, w)


def _row_chunks(total, step):
    sizes = []
    left = total
    while left > 0:
        sizes.append(min(step, left))
        left -= sizes[-1]
    return sizes


def _spmm_sc(src, dst, ew, sup0, sup1, b2):
    n, h = sup0.shape
    e = src.shape[0]
    e_tile = e // _NS
    nfull = e_tile // _CH
    rem = e_tile - nfull * _CH
    r_tile = n // _NS
    nvec = h // 16

    mesh = plsc.VectorSubcoreMesh(core_axis_name="c", subcore_axis_name="s")

    @functools.partial(
        pl.kernel,
        out_type=[jax.ShapeDtypeStruct((n, h), jnp.float32)] * 2,
        mesh=mesh,
        scratch_types=[
            pltpu.VMEM((_CH,), jnp.int32),
            pltpu.VMEM((_CH,), jnp.int32),
            pltpu.VMEM((_CH,), jnp.float32),
            pltpu.VMEM((_CH, h), jnp.float32),
            pltpu.VMEM((rem,), jnp.int32),
            pltpu.VMEM((rem,), jnp.int32),
            pltpu.VMEM((rem,), jnp.float32),
            pltpu.VMEM((rem, h), jnp.float32),
            pltpu.VMEM((h,), jnp.float32),
            pltpu.VMEM_SHARED((n, h), jnp.float32),
            pltpu.SemaphoreType.DMA,
        ],
    )
    def spmm(src_h, dst_h, ew_h, sup0_h, sup1_h, b2_h, out0_h, out1_h,
             sidx, didx, wv, rows, sidx_r, didx_r, wv_r, rows_r, bvec, acc,
             sem):
        c = lax.axis_index("c")
        s = lax.axis_index("s")

        def scale_rows(rows_ref, wv_ref, cnt):
            def body(i, carry):
                w = wv_ref[i]
                for j in range(nvec):
                    sl = (i, pl.ds(16 * j, 16))
                    rows_ref[sl] = rows_ref[sl] * w
                return carry
            lax.fori_loop(0, cnt, body, 0)

        def chunk(sup_h, base, c_sidx, c_didx, c_wv, c_rows, cnt):
            pltpu.sync_copy(src_h.at[pl.ds(base, cnt)], c_sidx)
            pltpu.sync_copy(dst_h.at[pl.ds(base, cnt)], c_didx)
            pltpu.sync_copy(ew_h.at[pl.ds(base, cnt)], c_wv)
            pltpu.async_copy(sup_h.at[c_sidx], c_rows, sem).wait()
            scale_rows(c_rows, c_wv, cnt)
            pltpu.sync_copy(c_rows, acc.at[c_didx], add=True)

        def run(ci, sup_h, out_h):
            zero = jnp.zeros((16,), jnp.float32)

            def zbody(i, carry):
                for j in range(nvec):
                    rows[i, pl.ds(16 * j, 16)] = zero
                return carry
            lax.fori_loop(0, _CH, zbody, 0)
            r0 = s * r_tile
            off = 0
            for sz in _row_chunks(r_tile, _CH):
                pltpu.sync_copy(rows.at[pl.ds(0, sz)],
                                acc.at[pl.ds(r0 + off, sz)])
                off += sz
            plsc.subcore_barrier()

            e0 = s * e_tile

            def ebody(k, carry):
                chunk(sup_h, e0 + k * _CH, sidx, didx, wv, rows, _CH)
                return carry
            lax.fori_loop(0, nfull, ebody, 0)
            if rem:
                chunk(sup_h, e0 + nfull * _CH, sidx_r, didx_r, wv_r, rows_r,
                      rem)
            plsc.subcore_barrier()

            pltpu.sync_copy(b2_h.at[ci], bvec)
            off = 0
            for sz in _row_chunks(r_tile, _CH):
                pltpu.sync_copy(acc.at[pl.ds(r0 + off, sz)],
                                rows.at[pl.ds(0, sz)])

                def bbody(i, carry):
                    for j in range(nvec):
                        sl = (i, pl.ds(16 * j, 16))
                        rows[sl] = rows[sl] + bvec[pl.ds(16 * j, 16)]
                    return carry
                lax.fori_loop(0, sz, bbody, 0)
                pltpu.sync_copy(rows.at[pl.ds(0, sz)],
                                out_h.at[pl.ds(r0 + off, sz)])
                off += sz

        @pl.when(c == 0)
        def _():
            run(0, sup0_h, out0_h)

        @pl.when(c == 1)
        def _():
            run(1, sup1_h, out1_h)

    return spmm(src, dst, ew, sup0, sup1, b2)


def kernel(edge_index, edge_weight, input_feature, W, b):
    src = edge_index[0]
    dst = edge_index[1]
    sup0, sup1 = _matmul_split(input_feature, W)
    b2 = b.reshape(2, -1)
    o0, o1 = _spmm_sc(src, dst, edge_weight, sup0, sup1, b2)
    return jnp.concatenate([o0, o1], axis=1)


# speedup vs baseline: 5.0759x; 5.0759x over previous
import functools

import jax
import jax.numpy as jnp
from jax import lax
from jax.experimental import pallas as pl
from jax.experimental.pallas import tpu as pltpu
from jax.experimental.pallas import tpu_sc as plsc

_NS = 16
_NC = 2
_CH = 128


def _matmul(x, w):
    n = x.shape[0]
    d = w.shape[1]

    def body(x_ref, w_ref, o_ref):
        o_ref[...] = jnp.dot(x_ref[...], w_ref[...],
                             preferred_element_type=jnp.float32)

    return pl.pallas_call(
        body,
        out_shape=jax.ShapeDtypeStruct((n, d), jnp.float32),
    )(x, w)


def _combine(p, b):
    _, n, d = p.shape
    blk = 2000

    def body(p_ref, b_ref, o_ref):
        o_ref[...] = p_ref[0] + p_ref[1] + b_ref[...]

    return pl.pallas_call(
        body,
        grid=(n // blk,),
        in_specs=[
            pl.BlockSpec((_NC, blk, d), lambda i: (0, i, 0)),
            pl.BlockSpec((1, d), lambda i: (0, 0)),
        ],
        out_specs=pl.BlockSpec((blk, d), lambda i: (i, 0)),
        out_shape=jax.ShapeDtypeStruct((n, d), jnp.float32),
    )(p, b.reshape(1, d))


def _row_chunks(total, step):
    sizes = []
    left = total
    while left > 0:
        sizes.append(min(step, left))
        left -= sizes[-1]
    return sizes


def _spmm_sc(src, dst, ew, sup):
    n, d = sup.shape
    e = src.shape[0]
    nw = _NC * _NS
    e_w = e // nw
    nfull = e_w // _CH
    rem = e_w - nfull * _CH
    r_tile = (n // _NS) // 8 * 8
    r_last = n - (_NS - 1) * r_tile
    nvec = d // 16

    mesh = plsc.VectorSubcoreMesh(core_axis_name="c", subcore_axis_name="s")

    scratch = [
        pltpu.VMEM((_CH,), jnp.int32),
        pltpu.VMEM((_CH,), jnp.int32),
        pltpu.VMEM((_CH,), jnp.float32),
        pltpu.VMEM((_CH, d), jnp.float32),
        pltpu.VMEM_SHARED((n, d), jnp.float32),
        pltpu.SemaphoreType.DMA,
    ]
    if rem:
        scratch = scratch[:4] + [
            pltpu.VMEM((rem,), jnp.int32),
            pltpu.VMEM((rem,), jnp.int32),
            pltpu.VMEM((rem,), jnp.float32),
            pltpu.VMEM((rem, d), jnp.float32),
        ] + scratch[4:]

    @functools.partial(
        pl.kernel,
        out_type=jax.ShapeDtypeStruct((_NC, n, d), jnp.float32),
        mesh=mesh,
        scratch_types=scratch,
    )
    def spmm(src_h, dst_h, ew_h, sup_h, out_h, sidx, didx, wv, rows,
             *rest):
        if rem:
            sidx_r, didx_r, wv_r, rows_r, acc, sem = rest
        else:
            acc, sem = rest
        c = lax.axis_index("c")
        s = lax.axis_index("s")
        wid = s * _NC + c

        def scale_rows(rows_ref, wv_ref, cnt):
            def body(g, carry):
                w16 = wv_ref[pl.ds(g * 16, 16)]
                for t in range(16):
                    w = w16[t]
                    i = g * 16 + t
                    for j in range(nvec):
                        sl = (i, pl.ds(16 * j, 16))
                        rows_ref[sl] = rows_ref[sl] * w
                return carry
            lax.fori_loop(0, cnt // 16, body, 0)

        def chunk(base, c_sidx, c_didx, c_wv, c_rows, cnt):
            pltpu.sync_copy(src_h.at[pl.ds(base, cnt)], c_sidx)
            pltpu.sync_copy(dst_h.at[pl.ds(base, cnt)], c_didx)
            pltpu.sync_copy(ew_h.at[pl.ds(base, cnt)], c_wv)
            pltpu.async_copy(sup_h.at[c_sidx], c_rows, sem).wait()
            scale_rows(c_rows, c_wv, cnt)
            pltpu.sync_copy(c_rows, acc.at[c_didx], add=True)

        def for_slab(fn):
            @pl.when(s < _NS - 1)
            def _():
                fn(s * r_tile, _row_chunks(r_tile, _CH))

            @pl.when(s == _NS - 1)
            def _():
                fn((_NS - 1) * r_tile, _row_chunks(r_last, _CH))

        zero = jnp.zeros((16,), jnp.float32)

        def zbody(i, carry):
            for j in range(nvec):
                rows[i, pl.ds(16 * j, 16)] = zero
            return carry
        lax.fori_loop(0, _CH, zbody, 0)

        def init_fn(r0, sizes):
            off = 0
            for sz in sizes:
                base = pl.multiple_of(r0 + off, 8)
                pltpu.sync_copy(rows.at[pl.ds(0, sz)], acc.at[pl.ds(base, sz)])
                off += sz
        for_slab(init_fn)
        plsc.subcore_barrier()

        e0 = wid * e_w

        def ebody(k, carry):
            chunk(e0 + k * _CH, sidx, didx, wv, rows, _CH)
            return carry
        lax.fori_loop(0, nfull, ebody, 0)
        if rem:
            chunk(e0 + nfull * _CH, sidx_r, didx_r, wv_r, rows_r, rem)
        plsc.subcore_barrier()

        def out_fn(r0, sizes):
            off = 0
            for sz in sizes:
                base = pl.multiple_of(r0 + off, 8)
                pltpu.sync_copy(acc.at[pl.ds(base, sz)], rows.at[pl.ds(0, sz)])
                pltpu.sync_copy(rows.at[pl.ds(0, sz)],
                                out_h.at[c, pl.ds(base, sz)])
                off += sz
        for_slab(out_fn)

    return spmm(src, dst, ew, sup)


def kernel(edge_index, edge_weight, input_feature, W, b):
    src = edge_index[0]
    dst = edge_index[1]
    sup = _matmul(input_feature, W)
    partials = _spmm_sc(src, dst, edge_weight, sup)
    return _combine(partials, b)
